# baseline (device time: 60868 ns/iter reference)
import contextlib
import os

import jax
import jax.numpy as jnp
from jax import lax
from jax.experimental import pallas as pl
from jax.experimental.pallas import tpu as pltpu

KMODE = os.environ.get("KMODE", "full")
KSCOPE = os.environ.get("KSCOPE", "0") == "1"


def _scope(name):
    return jax.named_scope(name) if KSCOPE else contextlib.nullcontext()


N_DEV = 16
B, SQ, DM = 2, 512, 768
DH = 64
BLK = 64
ROWS = B * SQ
CH = ROWS // N_DEV


def kernel(x, Wq, K_ext, V_ext, Wo):
    H = K_ext.shape[2]
    HD = H * DH

    idx = lax.axis_index("i")
    x2 = x.reshape(ROWS, DM)
    k2 = K_ext.reshape(ROWS, HD)
    v2 = V_ext.reshape(ROWS, HD)
    wq_s = lax.dynamic_slice(Wq, (0, idx * HD), (DM, HD)).astype(jnp.bfloat16)
    wo_s = lax.dynamic_slice(Wo, (idx * HD, 0), (HD, DM)).astype(jnp.bfloat16)

    def body(x_ref, wq_ref, k_ref, v_ref, wo_ref, o_ref,
             xb_ref, kb_ref, vb_ref, q_ref, ctx_ref, part_ref, red_ref,
             rs_recv, agbuf,
             rs_ssem, rs_rsem, ag_ssem, ag_rsem):
        me = lax.axis_index("i")

        if KMODE != "nocomm":
            bar = pltpu.get_barrier_semaphore()
            for j in range(N_DEV):
                pl.semaphore_signal(bar, inc=1, device_id=(j,),
                                    device_id_type=pl.DeviceIdType.MESH)

        with _scope("ph_prep"):
            xb_ref[...] = x_ref[...].astype(jnp.bfloat16)
            kb_ref[...] = k_ref[...].astype(jnp.bfloat16)
            vb_ref[...] = v_ref[...].astype(jnp.bfloat16)
            q_ref[...] = jnp.dot(
                xb_ref[...], wq_ref[...], preferred_element_type=jnp.float32
            ).astype(jnp.bfloat16)

        qb = lax.broadcasted_iota(jnp.int32, (SQ, SQ), 0) // BLK
        kb = lax.broadcasted_iota(jnp.int32, (SQ, SQ), 1) // BLK
        mask = (qb == kb) | (kb == 0) | ((qb + kb) % 3 == 0)
        bias = jnp.where(mask, 0.0, -1e9).astype(jnp.float32)

        with _scope("ph_attn"):
            for b in range(B):
                for h in range(H):
                    if KMODE == "noattn":
                        break
                    cols = slice(h * DH, (h + 1) * DH)
                    rows = slice(b * SQ, (b + 1) * SQ)
                    s = lax.dot_general(
                        q_ref[rows, cols], kb_ref[rows, cols],
                        (((1,), (1,)), ((), ())),
                        preferred_element_type=jnp.float32,
                    )
                    e = jnp.exp(s * 0.125 + bias)
                    rsum = jnp.sum(e, axis=1, keepdims=True)
                    ctx = jnp.dot(e.astype(jnp.bfloat16), vb_ref[rows, cols],
                                  preferred_element_type=jnp.float32)
                    ctx = ctx * (1.0 / rsum)
                    ctx_ref[rows, cols] = ctx.astype(jnp.bfloat16)

        if KMODE == "nocomm":
            for o in range(N_DEV):
                j = (me + o) % N_DEV
                rows = pl.ds(j * CH, CH)
                o_ref[rows, :] = jnp.dot(ctx_ref[rows, :], wo_ref[...],
                                         preferred_element_type=jnp.float32)
            return

        with _scope("ph_barrier"):
            pl.semaphore_wait(bar, N_DEV)

        if KMODE == "justbar":
            for o in range(N_DEV):
                j = (me + o) % N_DEV
                rows = pl.ds(j * CH, CH)
                o_ref[rows, :] = jnp.dot(ctx_ref[rows, :], wo_ref[...],
                                         preferred_element_type=jnp.float32)
            return

        with _scope("ph_rs"):
            rs_descs = []
            for o in range(N_DEV):
                j = (me + o) % N_DEV
                rows = pl.ds(j * CH, CH)
                pc = jnp.dot(ctx_ref[rows, :], wo_ref[...],
                             preferred_element_type=jnp.float32
                             ).astype(jnp.bfloat16)
                part_ref[rows, :] = pc
                if o == 0:
                    rs_recv[me, :, :] = pc
                else:
                    d = pltpu.make_async_remote_copy(
                        src_ref=part_ref.at[rows, :],
                        dst_ref=rs_recv.at[me],
                        send_sem=rs_ssem.at[o - 1],
                        recv_sem=rs_rsem.at[o - 1],
                        device_id=(j,), device_id_type=pl.DeviceIdType.MESH)
                    d.start()
                    rs_descs.append(d)

        with _scope("ph_reduce"):
            acc = rs_recv[me].astype(jnp.float32)
            for o in range(1, N_DEV):
                rs_descs[o - 1].wait_recv()
                acc = acc + rs_recv[(me - o) % N_DEV].astype(jnp.float32)
            red_ref[...] = acc.astype(jnp.bfloat16)

        with _scope("ph_ag"):
            myrows = pl.ds(me * CH, CH)
            agbuf[myrows, :] = red_ref[...]
            o_ref[myrows, :] = acc
            ag_descs = []
            for o in range(1, N_DEV):
                j = (me + o) % N_DEV
                d = pltpu.make_async_remote_copy(
                    src_ref=red_ref,
                    dst_ref=agbuf.at[myrows, :],
                    send_sem=ag_ssem.at[o - 1],
                    recv_sem=ag_rsem.at[o - 1],
                    device_id=(j,), device_id_type=pl.DeviceIdType.MESH)
                d.start()
                ag_descs.append(d)
        with _scope("ph_drain_rs"):
            for d in rs_descs:
                d.wait_send()
        with _scope("ph_collect"):
            for o in range(1, N_DEV):
                ag_descs[o - 1].wait_recv()
                rows = pl.ds(((me - o) % N_DEV) * CH, CH)
                o_ref[rows, :] = agbuf[rows, :].astype(jnp.float32)
        with _scope("ph_drain_ag"):
            for d in ag_descs:
                d.wait_send()

    out = pl.pallas_call(
        body,
        out_shape=jax.ShapeDtypeStruct((ROWS, DM), jnp.float32),
        in_specs=[pl.BlockSpec(memory_space=pltpu.VMEM)] * 5,
        out_specs=pl.BlockSpec(memory_space=pltpu.VMEM),
        scratch_shapes=[
            pltpu.VMEM((ROWS, DM), jnp.bfloat16),
            pltpu.VMEM((ROWS, HD), jnp.bfloat16),
            pltpu.VMEM((ROWS, HD), jnp.bfloat16),
            pltpu.VMEM((ROWS, HD), jnp.bfloat16),
            pltpu.VMEM((ROWS, HD), jnp.bfloat16),
            pltpu.VMEM((ROWS, DM), jnp.bfloat16),
            pltpu.VMEM((CH, DM), jnp.bfloat16),
            pltpu.VMEM((N_DEV, CH, DM), jnp.bfloat16),
            pltpu.VMEM((ROWS, DM), jnp.bfloat16),
            pltpu.SemaphoreType.DMA((N_DEV - 1,)),
            pltpu.SemaphoreType.DMA((N_DEV - 1,)),
            pltpu.SemaphoreType.DMA((N_DEV - 1,)),
            pltpu.SemaphoreType.DMA((N_DEV - 1,)),
        ],
        compiler_params=pltpu.CompilerParams(
            collective_id=None if KMODE == "nocomm" else 0),
    )(x2, wq_s, k2, v2, wo_s)

    return out.reshape(B, SQ, DM)
